# baseline (speedup 1.0000x reference)
import functools

import jax
import jax.numpy as jnp
from jax import lax
from jax.experimental import pallas as pl
from jax.experimental.pallas import tpu as pltpu
from jax.experimental.pallas import tpu_sc as plsc

NUM_TICKERS = 1000
EMBED_DIM = 32
LANES = 128
BATCH = 16384
HALF = BATCH // 2

_INFO = plsc.get_sparse_core_info()
_NC = _INFO.num_cores
_NS = _INFO.num_subcores
_NW = _NC * _NS

_MESH = plsc.VectorSubcoreMesh(core_axis_name="c", subcore_axis_name="s")


@functools.lru_cache(maxsize=None)
def _make_gather(batch):
    b_per_w = batch // _NW
    idx_rows = batch // LANES

    @functools.partial(
        pl.kernel,
        mesh=_MESH,
        out_type=jax.ShapeDtypeStruct((batch, LANES), jnp.float32),
        scratch_types=[
            pltpu.VMEM((idx_rows // _NW, LANES), jnp.int32),
            pltpu.VMEM((b_per_w, EMBED_DIM), jnp.float32),
            pltpu.SemaphoreType.DMA,
        ],
        compiler_params=pltpu.CompilerParams(use_tc_tiling_on_sc=False),
    )
    def _embed_gather(tickers_hbm, table_hbm, out_hbm, idx_v, rows_v, sem):
        wid = lax.axis_index("s") * _NC + lax.axis_index("c")
        rows_per_w = idx_rows // _NW
        pltpu.sync_copy(tickers_hbm.at[pl.ds(wid * rows_per_w, rows_per_w)], idx_v)
        gathers = [
            pltpu.async_copy(
                table_hbm.at[idx_v.at[j]],
                rows_v.at[pl.ds(j * LANES, LANES)],
                sem,
            )
            for j in range(rows_per_w)
        ]
        for g in gathers:
            g.wait()
        pltpu.sync_copy(
            rows_v,
            out_hbm.at[pl.ds(wid * b_per_w, b_per_w), pl.ds(0, EMBED_DIM)],
        )

    return _embed_gather


def kernel(tickers, table):
    t128 = tickers.astype(jnp.int32).reshape(BATCH // LANES, LANES)
    gather = _make_gather(HALF)
    rows = BATCH // LANES // 2
    pa = gather(t128[:rows], table)
    pb = gather(t128[rows:], table)
    return jnp.concatenate([pa[:, :EMBED_DIM], pb[:, :EMBED_DIM]], axis=0)


# speedup vs baseline: 1.1117x; 1.1117x over previous
import functools

import jax
import jax.numpy as jnp
from jax import lax
from jax.experimental import pallas as pl
from jax.experimental.pallas import tpu as pltpu
from jax.experimental.pallas import tpu_sc as plsc

NUM_TICKERS = 1000
EMBED_DIM = 32
LANES = 128
BATCH = 16384

_INFO = plsc.get_sparse_core_info()
_NC = _INFO.num_cores
_NS = _INFO.num_subcores
_NW = _NC * _NS
_B_PER_W = BATCH // _NW

_MESH = plsc.VectorSubcoreMesh(core_axis_name="c", subcore_axis_name="s")


@functools.partial(
    pl.kernel,
    mesh=_MESH,
    out_type=jax.ShapeDtypeStruct((BATCH, LANES), jnp.float32),
    scratch_types=[
        pltpu.VMEM((_B_PER_W,), jnp.int32),
        pltpu.VMEM((_B_PER_W, EMBED_DIM), jnp.float32),
        pltpu.SemaphoreType.DMA,
    ],
    compiler_params=pltpu.CompilerParams(use_tc_tiling_on_sc=False),
)
def _embed_gather(tickers_hbm, table_hbm, out_hbm, idx_v, rows_v, sem):
    wid = lax.axis_index("s") * _NC + lax.axis_index("c")
    base = wid * _B_PER_W
    pltpu.sync_copy(tickers_hbm.at[pl.ds(base, _B_PER_W)], idx_v)
    pltpu.async_copy(table_hbm.at[idx_v], rows_v, sem).wait()
    pltpu.sync_copy(rows_v, out_hbm.at[pl.ds(base, _B_PER_W), pl.ds(0, EMBED_DIM)])


def kernel(tickers, table):
    padded = _embed_gather(tickers.astype(jnp.int32), table)
    return padded[:, :EMBED_DIM]
